# baseline (speedup 1.0000x reference)
import functools

import jax
import jax.numpy as jnp
from jax import lax
from jax.experimental import pallas as pl
from jax.experimental.pallas import tpu as pltpu
from jax.experimental.pallas import tpu_sc as plsc

DIM = 2048
NE = 16
K = 2
NT = 4 * 4096
TILE = 2048
L = 16
NC = 2
NS = 16
NW = NC * NS
TOK_W = NT // NW


WPT = TILE // TOK_W


def _logits_body(x_ref, wt_ref, b_ref, out_ref):
    acc = jnp.dot(x_ref[...], wt_ref[...], preferred_element_type=jnp.float32)
    acc = acc + b_ref[...]
    out_ref[...] = acc.reshape(WPT, TOK_W, NE).transpose(0, 2, 1)


def _compute_logits_blocked(xf, wt, bias2d):
    return pl.pallas_call(
        _logits_body,
        grid=(NT // TILE,),
        in_specs=[
            pl.BlockSpec((TILE, DIM), lambda i: (i, 0)),
            pl.BlockSpec(memory_space=pltpu.VMEM),
            pl.BlockSpec(memory_space=pltpu.VMEM),
        ],
        out_specs=pl.BlockSpec((WPT, NE, TOK_W), lambda i: (i, 0, 0)),
        out_shape=jax.ShapeDtypeStruct((NW, NE, TOK_W), jnp.float32),
        compiler_params=pltpu.CompilerParams(
            dimension_semantics=("parallel",),
        ),
    )(xf, wt, bias2d)


_sc_mesh = plsc.VectorSubcoreMesh(core_axis_name="c", subcore_axis_name="s")


@functools.partial(
    pl.kernel,
    mesh=_sc_mesh,
    out_type=[
        jax.ShapeDtypeStruct((NW, K, TOK_W), jnp.float32),
        jax.ShapeDtypeStruct((NW, K, TOK_W), jnp.int32),
    ],
    scratch_types=[
        pltpu.VMEM((NE, TOK_W), jnp.float32),
        pltpu.VMEM((K, TOK_W), jnp.float32),
        pltpu.VMEM((K, TOK_W), jnp.int32),
    ],
)
def _route_sc(lg_hbm, p_hbm, i_hbm, lg_v, p_v, i_v):
    wid = lax.axis_index("s") * NC + lax.axis_index("c")
    pltpu.sync_copy(lg_hbm.at[wid], lg_v)

    def body(g, carry):
        off = g * L
        m1 = lg_v[0, pl.ds(off, L)]
        i1 = jnp.zeros((L,), jnp.int32)
        m2 = jnp.full((L,), -jnp.inf, jnp.float32)
        i2 = jnp.zeros((L,), jnp.int32)
        for e in range(1, NE):
            v = lg_v[e, pl.ds(off, L)]
            gt1 = v > m1
            gt2 = v > m2
            e_v = jnp.full((L,), e, jnp.int32)
            m2 = jnp.where(gt1, m1, jnp.where(gt2, v, m2))
            i2 = jnp.where(gt1, i1, jnp.where(gt2, e_v, i2))
            m1 = jnp.where(gt1, v, m1)
            i1 = jnp.where(gt1, e_v, i1)
        t = jnp.exp(m2 - m1)
        p1 = 1.0 / (1.0 + t)
        p_v[0, pl.ds(off, L)] = p1
        p_v[1, pl.ds(off, L)] = 1.0 - p1
        i_v[0, pl.ds(off, L)] = i1
        i_v[1, pl.ds(off, L)] = i2
        return carry

    lax.fori_loop(0, TOK_W // L, body, 0, unroll=False)
    pltpu.sync_copy(p_v, p_hbm.at[wid])
    pltpu.sync_copy(i_v, i_hbm.at[wid])


def kernel(x, W, expert_bias, ema_load):
    b, s, d = x.shape
    xf = x.reshape(b * s, d)
    wt = W.T
    bias2d = expert_bias.reshape(1, NE)
    logits_blk = _compute_logits_blocked(xf, wt, bias2d)
    p_blk, i_blk = _route_sc(logits_blk)
    topk_prob = p_blk.transpose(0, 2, 1).reshape(b, s, K)
    topk_idx = i_blk.transpose(0, 2, 1).reshape(b, s, K)
    return (topk_prob, topk_idx)


# speedup vs baseline: 1.0269x; 1.0269x over previous
import functools

import jax
import jax.numpy as jnp
from jax import lax
from jax.experimental import pallas as pl
from jax.experimental.pallas import tpu as pltpu
from jax.experimental.pallas import tpu_sc as plsc

DIM = 2048
NE = 16
K = 2
NT = 4 * 4096
TILE = 1024
L = 16
NC = 2
NS = 16
NW = NC * NS
TOK_W = NT // NW


WPT = TILE // TOK_W


def _logits_body(x_ref, wt_ref, b_ref, out_ref):
    acc = jnp.dot(x_ref[...], wt_ref[...], preferred_element_type=jnp.float32)
    acc = acc + b_ref[...]
    out_ref[...] = acc.reshape(WPT, TOK_W, NE).transpose(0, 2, 1)


def _compute_logits_blocked(xf, wt, bias2d):
    return pl.pallas_call(
        _logits_body,
        grid=(NT // TILE,),
        in_specs=[
            pl.BlockSpec((TILE, DIM), lambda i: (i, 0)),
            pl.BlockSpec(memory_space=pltpu.VMEM),
            pl.BlockSpec(memory_space=pltpu.VMEM),
        ],
        out_specs=pl.BlockSpec((WPT, NE, TOK_W), lambda i: (i, 0, 0)),
        out_shape=jax.ShapeDtypeStruct((NW, NE, TOK_W), jnp.float32),
        compiler_params=pltpu.CompilerParams(
            dimension_semantics=("parallel",),
        ),
    )(xf, wt, bias2d)


_sc_mesh = plsc.VectorSubcoreMesh(core_axis_name="c", subcore_axis_name="s")


@functools.partial(
    pl.kernel,
    mesh=_sc_mesh,
    out_type=[
        jax.ShapeDtypeStruct((NW, K, TOK_W), jnp.float32),
        jax.ShapeDtypeStruct((NW, K, TOK_W), jnp.int32),
    ],
    scratch_types=[
        pltpu.VMEM((NE, TOK_W), jnp.float32),
        pltpu.VMEM((K, TOK_W), jnp.float32),
        pltpu.VMEM((K, TOK_W), jnp.int32),
    ],
)
def _route_sc(lg_hbm, p_hbm, i_hbm, lg_v, p_v, i_v):
    wid = lax.axis_index("s") * NC + lax.axis_index("c")
    pltpu.sync_copy(lg_hbm.at[wid], lg_v)

    def body(g, carry):
        off = g * L
        m1 = lg_v[0, pl.ds(off, L)]
        i1 = jnp.zeros((L,), jnp.int32)
        m2 = jnp.full((L,), -jnp.inf, jnp.float32)
        i2 = jnp.zeros((L,), jnp.int32)
        for e in range(1, NE):
            v = lg_v[e, pl.ds(off, L)]
            gt1 = v > m1
            gt2 = v > m2
            e_v = jnp.full((L,), e, jnp.int32)
            m2 = jnp.where(gt1, m1, jnp.where(gt2, v, m2))
            i2 = jnp.where(gt1, i1, jnp.where(gt2, e_v, i2))
            m1 = jnp.where(gt1, v, m1)
            i1 = jnp.where(gt1, e_v, i1)
        t = jnp.exp(m2 - m1)
        p1 = 1.0 / (1.0 + t)
        p_v[0, pl.ds(off, L)] = p1
        p_v[1, pl.ds(off, L)] = 1.0 - p1
        i_v[0, pl.ds(off, L)] = i1
        i_v[1, pl.ds(off, L)] = i2
        return carry

    lax.fori_loop(0, TOK_W // L, body, 0, unroll=False)
    pltpu.sync_copy(p_v, p_hbm.at[wid])
    pltpu.sync_copy(i_v, i_hbm.at[wid])


def kernel(x, W, expert_bias, ema_load):
    b, s, d = x.shape
    xf = x.reshape(b * s, d)
    wt = W.T
    bias2d = expert_bias.reshape(1, NE)
    logits_blk = _compute_logits_blocked(xf, wt, bias2d)
    p_blk, i_blk = _route_sc(logits_blk)
    topk_prob = p_blk.transpose(0, 2, 1).reshape(b, s, K)
    topk_idx = i_blk.transpose(0, 2, 1).reshape(b, s, K)
    return (topk_prob, topk_idx)
